# Initial kernel scaffold; baseline (speedup 1.0000x reference)
import functools

import jax
import jax.numpy as jnp
from jax import lax
from jax.experimental import pallas as pl
from jax.experimental.pallas import tpu as pltpu
from jax.experimental.pallas import tpu_sc as plsc

N = 10000
D = 128
E = 320000

CHUNK = 128
NCHUNKS = E // CHUNK
NWORKERS = 32
CHUNKS_PER_TILE = (NCHUNKS + NWORKERS - 1) // NWORKERS
ROWS_PER_TILE = N // 16


@functools.partial(
    pl.kernel,
    mesh=plsc.VectorSubcoreMesh(core_axis_name="c", subcore_axis_name="s"),
    out_type=jax.ShapeDtypeStruct((2, N, D), jnp.float32),
    scratch_types=[
        pltpu.VMEM((CHUNK,), jnp.int32),
        pltpu.VMEM((CHUNK,), jnp.int32),
        pltpu.VMEM((CHUNK,), jnp.float32),
        pltpu.VMEM((CHUNK, D), jnp.float32),
        pltpu.VMEM_SHARED((N, D), jnp.float32),
        pltpu.SemaphoreType.DMA,
    ],
)
def _sc_aggregate(src_hbm, dst_hbm, w_hbm, ego_hbm, zeros_hbm, out_hbm,
                  src_v, dst_v, w_v, rows_v, acc, sem):
    c = lax.axis_index("c")
    s = lax.axis_index("s")
    wid = s * 2 + c

    row0 = s * ROWS_PER_TILE
    pltpu.sync_copy(zeros_hbm.at[pl.ds(row0, ROWS_PER_TILE)],
                    acc.at[pl.ds(row0, ROWS_PER_TILE)])
    plsc.subcore_barrier()

    def chunk_body(k, carry):
        chunk_id = k * NWORKERS + wid

        @pl.when(chunk_id < NCHUNKS)
        def _():
            off = chunk_id * CHUNK
            pltpu.sync_copy(src_hbm.at[pl.ds(off, CHUNK)], src_v)
            pltpu.sync_copy(dst_hbm.at[pl.ds(off, CHUNK)], dst_v)
            pltpu.sync_copy(w_hbm.at[pl.ds(off, CHUNK)], w_v)
            pltpu.async_copy(ego_hbm.at[src_v], rows_v, sem).wait()

            def edge_body(e, carry2):
                w = w_v[e]
                for j in range(D // 16):
                    sl = pl.ds(j * 16, 16)
                    rows_v[e, sl] = rows_v[e, sl] * w
                return carry2

            lax.fori_loop(0, CHUNK, edge_body, 0)
            pltpu.sync_copy(rows_v, acc.at[dst_v], add=True)

        return carry

    lax.fori_loop(0, CHUNKS_PER_TILE, chunk_body, 0)
    plsc.subcore_barrier()
    pltpu.sync_copy(acc.at[pl.ds(row0, ROWS_PER_TILE)],
                    out_hbm.at[c, pl.ds(row0, ROWS_PER_TILE)])


ROWS_BLK = 1000


def _tc_dense_body(ego_ref, p_ref, w1_ref, b1_ref, w2_ref, b2_ref, out_ref):
    side = p_ref[0] + p_ref[1]
    ego = ego_ref[...]
    dn = (((1,), (1,)), ((), ()))
    y1 = lax.dot_general(ego + side, w1_ref[...], dn,
                         preferred_element_type=jnp.float32) + b1_ref[...]
    y1 = jnp.where(y1 >= 0, y1, 0.01 * y1)
    y2 = lax.dot_general(ego * side, w2_ref[...], dn,
                         preferred_element_type=jnp.float32) + b2_ref[...]
    y2 = jnp.where(y2 >= 0, y2, 0.01 * y2)
    out_ref[...] = y1 + y2


_tc_dense = pl.pallas_call(
    _tc_dense_body,
    grid=(N // ROWS_BLK,),
    in_specs=[
        pl.BlockSpec((ROWS_BLK, D), lambda i: (i, 0)),
        pl.BlockSpec((2, ROWS_BLK, D), lambda i: (0, i, 0)),
        pl.BlockSpec((D, D), lambda i: (0, 0)),
        pl.BlockSpec((1, D), lambda i: (0, 0)),
        pl.BlockSpec((D, D), lambda i: (0, 0)),
        pl.BlockSpec((1, D), lambda i: (0, 0)),
    ],
    out_specs=pl.BlockSpec((ROWS_BLK, D), lambda i: (i, 0)),
    out_shape=jax.ShapeDtypeStruct((N, D), jnp.float32),
)


def kernel(ego_embeddings, edge_index, edge_weight, W1, b1, W2, b2):
    src = edge_index[0].astype(jnp.int32)
    dst = edge_index[1].astype(jnp.int32)
    zeros = jnp.zeros((N, D), jnp.float32)
    partials = _sc_aggregate(src, dst, edge_weight, ego_embeddings, zeros)
    return _tc_dense(ego_embeddings, partials, W1, b1.reshape(1, D),
                     W2, b2.reshape(1, D))


# speedup vs baseline: 5.4351x; 5.4351x over previous
import functools

import jax
import jax.numpy as jnp
from jax import lax
from jax.experimental import pallas as pl
from jax.experimental.pallas import tpu as pltpu
from jax.experimental.pallas import tpu_sc as plsc

N = 10000
D = 128
E = 320000

CHUNK = 128
NCHUNKS = E // CHUNK
NWORKERS = 32
CHUNKS_PER_TILE = (NCHUNKS + NWORKERS - 1) // NWORKERS
ACC_ROWS = 10240
ROWS_PER_TILE = ACC_ROWS // 16


@functools.partial(
    pl.kernel,
    mesh=plsc.VectorSubcoreMesh(core_axis_name="c", subcore_axis_name="s"),
    out_type=jax.ShapeDtypeStruct((2, ACC_ROWS, D), jnp.float32),
    scratch_types=[
        pltpu.VMEM((CHUNK,), jnp.int32),
        pltpu.VMEM((CHUNK,), jnp.int32),
        pltpu.VMEM((CHUNK,), jnp.float32),
        pltpu.VMEM((CHUNK, D), jnp.float32),
        pltpu.VMEM_SHARED((ACC_ROWS, D), jnp.float32),
        pltpu.SemaphoreType.DMA,
    ],
)
def _sc_aggregate(src_hbm, dst_hbm, w_hbm, ego_hbm, zeros_hbm, out_hbm,
                  src_v, dst_v, w_v, rows_v, acc, sem):
    c = lax.axis_index("c")
    s = lax.axis_index("s")
    wid = s * 2 + c

    row0 = s * ROWS_PER_TILE
    pltpu.sync_copy(zeros_hbm, acc.at[pl.ds(row0, ROWS_PER_TILE)])
    plsc.subcore_barrier()

    def chunk_body(k, carry):
        chunk_id = k * NWORKERS + wid

        @pl.when(chunk_id < NCHUNKS)
        def _():
            off = chunk_id * CHUNK
            pltpu.sync_copy(src_hbm.at[pl.ds(off, CHUNK)], src_v)
            pltpu.sync_copy(dst_hbm.at[pl.ds(off, CHUNK)], dst_v)
            pltpu.sync_copy(w_hbm.at[pl.ds(off, CHUNK)], w_v)
            pltpu.async_copy(ego_hbm.at[src_v], rows_v, sem).wait()

            def group_body(g, carry2):
                w16 = w_v[pl.ds(g * 16, 16)]
                for lane in range(16):
                    e = g * 16 + lane
                    w = jnp.full((16,), w16[lane])
                    for j in range(D // 16):
                        sl = pl.ds(j * 16, 16)
                        rows_v[e, sl] = rows_v[e, sl] * w
                return carry2

            lax.fori_loop(0, CHUNK // 16, group_body, 0)
            pltpu.sync_copy(rows_v, acc.at[dst_v], add=True)

        return carry

    lax.fori_loop(0, CHUNKS_PER_TILE, chunk_body, 0)
    plsc.subcore_barrier()
    pltpu.sync_copy(acc.at[pl.ds(row0, ROWS_PER_TILE)],
                    out_hbm.at[c, pl.ds(row0, ROWS_PER_TILE)])


ROWS_BLK = 1000


def _tc_dense_body(ego_ref, p_ref, w1_ref, b1_ref, w2_ref, b2_ref, out_ref):
    side = p_ref[0] + p_ref[1]
    ego = ego_ref[...]
    dn = (((1,), (1,)), ((), ()))
    y1 = lax.dot_general(ego + side, w1_ref[...], dn,
                         preferred_element_type=jnp.float32) + b1_ref[...]
    y1 = jnp.where(y1 >= 0, y1, 0.01 * y1)
    y2 = lax.dot_general(ego * side, w2_ref[...], dn,
                         preferred_element_type=jnp.float32) + b2_ref[...]
    y2 = jnp.where(y2 >= 0, y2, 0.01 * y2)
    out_ref[...] = y1 + y2


_tc_dense = pl.pallas_call(
    _tc_dense_body,
    grid=(N // ROWS_BLK,),
    in_specs=[
        pl.BlockSpec((ROWS_BLK, D), lambda i: (i, 0)),
        pl.BlockSpec((2, ROWS_BLK, D), lambda i: (0, i, 0)),
        pl.BlockSpec((D, D), lambda i: (0, 0)),
        pl.BlockSpec((1, D), lambda i: (0, 0)),
        pl.BlockSpec((D, D), lambda i: (0, 0)),
        pl.BlockSpec((1, D), lambda i: (0, 0)),
    ],
    out_specs=pl.BlockSpec((ROWS_BLK, D), lambda i: (i, 0)),
    out_shape=jax.ShapeDtypeStruct((N, D), jnp.float32),
)


def kernel(ego_embeddings, edge_index, edge_weight, W1, b1, W2, b2):
    src = edge_index[0].astype(jnp.int32)
    dst = edge_index[1].astype(jnp.int32)
    zeros = jnp.zeros((ROWS_PER_TILE, D), jnp.float32)
    partials = _sc_aggregate(src, dst, edge_weight, ego_embeddings, zeros)
    return _tc_dense(ego_embeddings, partials, W1, b1.reshape(1, D),
                     W2, b2.reshape(1, D))


# speedup vs baseline: 8.9810x; 1.6524x over previous
import functools

import jax
import jax.numpy as jnp
from jax import lax
from jax.experimental import pallas as pl
from jax.experimental.pallas import tpu as pltpu
from jax.experimental.pallas import tpu_sc as plsc

N = 10000
D = 128
E = 320000

CHUNK = 128
NCHUNKS = E // CHUNK
NWORKERS = 32
CHUNKS_PER_TILE = (NCHUNKS + NWORKERS - 1) // NWORKERS
ACC_ROWS = 10240
ROWS_PER_TILE = ACC_ROWS // 16


PAIRS = (CHUNKS_PER_TILE + 1) // 2


@functools.partial(
    pl.kernel,
    mesh=plsc.VectorSubcoreMesh(core_axis_name="c", subcore_axis_name="s"),
    out_type=jax.ShapeDtypeStruct((2, ACC_ROWS, D), jnp.float32),
    scratch_types=[
        pltpu.VMEM((2, CHUNK), jnp.int32),
        pltpu.VMEM((2, CHUNK), jnp.int32),
        pltpu.VMEM((CHUNK,), jnp.float32),
        pltpu.VMEM((CHUNK,), jnp.float32),
        pltpu.VMEM((CHUNK, D), jnp.float32),
        pltpu.VMEM((CHUNK, D), jnp.float32),
        pltpu.VMEM_SHARED((ACC_ROWS, D), jnp.float32),
        pltpu.SemaphoreType.DMA,
        pltpu.SemaphoreType.DMA,
    ],
)
def _sc_aggregate(meta_hbm, w_hbm, ego_hbm, zeros_hbm, out_hbm,
                  meta0, meta1, w0, w1, rows0, rows1, acc, gsem0, gsem1):
    c = lax.axis_index("c")
    s = lax.axis_index("s")
    wid = s * 2 + c
    metas = (meta0, meta1)
    ws = (w0, w1)
    rows = (rows0, rows1)
    gsems = (gsem0, gsem1)

    row0 = s * ROWS_PER_TILE
    pltpu.sync_copy(zeros_hbm, acc.at[pl.ds(row0, ROWS_PER_TILE)])
    plsc.subcore_barrier()

    def fire(b, chunk_id):
        pltpu.sync_copy(meta_hbm.at[chunk_id], metas[b])
        pltpu.sync_copy(w_hbm.at[pl.ds(chunk_id * CHUNK, CHUNK)], ws[b])
        pltpu.make_async_copy(ego_hbm.at[metas[b].at[0]], rows[b],
                              gsems[b]).start()

    def process(b):
        pltpu.make_async_copy(ego_hbm.at[metas[b].at[0]], rows[b],
                              gsems[b]).wait()

        def group_body(g, carry2):
            w16 = ws[b][pl.ds(g * 16, 16)]
            for lane in range(16):
                e = g * 16 + lane
                w = jnp.full((16,), w16[lane])
                for j in range(D // 16):
                    sl = pl.ds(j * 16, 16)
                    rows[b][e, sl] = rows[b][e, sl] * w
            return carry2

        lax.fori_loop(0, CHUNK // 16, group_body, 0)
        pltpu.sync_copy(rows[b], acc.at[metas[b].at[1]], add=True)

    fire(0, wid)

    def pair_body(p, carry):
        for b in range(2):
            k = p * 2 + b
            cur = k * NWORKERS + wid

            @pl.when(cur < NCHUNKS)
            def _():
                nxt = cur + NWORKERS

                @pl.when(nxt < NCHUNKS)
                def _():
                    fire(1 - b, nxt)

                process(b)

        return carry

    lax.fori_loop(0, PAIRS, pair_body, 0)
    plsc.subcore_barrier()
    pltpu.sync_copy(acc.at[pl.ds(row0, ROWS_PER_TILE)],
                    out_hbm.at[c, pl.ds(row0, ROWS_PER_TILE)])


ROWS_BLK = 1000


def _tc_dense_body(ego_ref, p_ref, w1_ref, b1_ref, w2_ref, b2_ref, out_ref):
    side = p_ref[0] + p_ref[1]
    ego = ego_ref[...]
    dn = (((1,), (1,)), ((), ()))
    y1 = lax.dot_general(ego + side, w1_ref[...], dn,
                         preferred_element_type=jnp.float32) + b1_ref[...]
    y1 = jnp.where(y1 >= 0, y1, 0.01 * y1)
    y2 = lax.dot_general(ego * side, w2_ref[...], dn,
                         preferred_element_type=jnp.float32) + b2_ref[...]
    y2 = jnp.where(y2 >= 0, y2, 0.01 * y2)
    out_ref[...] = y1 + y2


_tc_dense = pl.pallas_call(
    _tc_dense_body,
    grid=(N // ROWS_BLK,),
    in_specs=[
        pl.BlockSpec((ROWS_BLK, D), lambda i: (i, 0)),
        pl.BlockSpec((2, ROWS_BLK, D), lambda i: (0, i, 0)),
        pl.BlockSpec((D, D), lambda i: (0, 0)),
        pl.BlockSpec((1, D), lambda i: (0, 0)),
        pl.BlockSpec((D, D), lambda i: (0, 0)),
        pl.BlockSpec((1, D), lambda i: (0, 0)),
    ],
    out_specs=pl.BlockSpec((ROWS_BLK, D), lambda i: (i, 0)),
    out_shape=jax.ShapeDtypeStruct((N, D), jnp.float32),
)


def kernel(ego_embeddings, edge_index, edge_weight, W1, b1, W2, b2):
    src = edge_index[0].astype(jnp.int32)
    dst = edge_index[1].astype(jnp.int32)
    meta = (jnp.stack([src, dst])
            .reshape(2, NCHUNKS, CHUNK)
            .transpose(1, 0, 2))
    zeros = jnp.zeros((ROWS_PER_TILE, D), jnp.float32)
    partials = _sc_aggregate(meta, edge_weight, ego_embeddings, zeros)
    return _tc_dense(ego_embeddings, partials, W1, b1.reshape(1, D),
                     W2, b2.reshape(1, D))
